# Initial kernel scaffold; baseline (speedup 1.0000x reference)
import jax
import jax.numpy as jnp
from jax.experimental import pallas as pl


def kernel(x, table):
    raise NotImplementedError("write your pallas kernel here")


# speedup vs baseline: 6.3260x; 6.3260x over previous
import functools

import jax
import jax.numpy as jnp
from jax import lax
from jax.experimental import pallas as pl
from jax.experimental.pallas import tpu as pltpu
from jax.experimental.pallas import tpu_sc as plsc

VOCAB = 100000
EMBED_DIM = 128
BATCH = 4096
SEQ_LEN = 200

NC = 2
NS = 16
NW = NC * NS

TOTAL = BATCH * SEQ_LEN
PER_W = TOTAL // NW
CHUNK = 128
NSTEPS = PER_W // CHUNK


def _body(x_hbm, table_hbm, out_hbm, idx_v, rows_v, sem):
    wid = lax.axis_index("s") * NC + lax.axis_index("c")
    base = wid * PER_W
    pltpu.sync_copy(x_hbm.at[wid], idx_v)

    def step(j):
        pltpu.async_copy(table_hbm.at[idx_v.at[j]], rows_v, sem).wait()
        pltpu.sync_copy(rows_v, out_hbm.at[pl.ds(base + j * CHUNK, CHUNK)])

    pl.loop(0, NSTEPS)(step)


@jax.jit
def kernel(x, table):
    x3 = x.reshape(NW, NSTEPS, CHUNK).astype(jnp.int32)
    run = functools.partial(
        pl.kernel,
        out_type=jax.ShapeDtypeStruct((TOTAL, EMBED_DIM), jnp.float32),
        mesh=plsc.VectorSubcoreMesh(core_axis_name="c", subcore_axis_name="s"),
        scratch_types=[
            pltpu.VMEM((NSTEPS, CHUNK), jnp.int32),
            pltpu.VMEM((CHUNK, EMBED_DIM), jnp.float32),
            pltpu.SemaphoreType.DMA,
        ],
    )(_body)
    out = run(x3, table)
    return out.reshape(BATCH, SEQ_LEN, EMBED_DIM)


# speedup vs baseline: 9.2734x; 1.4659x over previous
import functools

import jax
import jax.numpy as jnp
from jax import lax
from jax.experimental import pallas as pl
from jax.experimental.pallas import tpu as pltpu
from jax.experimental.pallas import tpu_sc as plsc

VOCAB = 100000
EMBED_DIM = 128
BATCH = 4096
SEQ_LEN = 200

NC = 2
NS = 16
NW = NC * NS

TOTAL = BATCH * SEQ_LEN
PER_W = TOTAL // NW
CHUNK = 128
NSTEPS = PER_W // CHUNK


NBUF = 4


def _body(x_hbm, table_hbm, out_hbm, idx_v, rows, sg, sw):
    wid = lax.axis_index("s") * NC + lax.axis_index("c")
    base = wid * PER_W
    pltpu.sync_copy(x_hbm.at[wid], idx_v)

    def gather(j, b):
        pltpu.async_copy(table_hbm.at[idx_v.at[j]], rows[b], sg[b])

    def write(j, b):
        return pltpu.async_copy(
            rows[b], out_hbm.at[pl.ds(base + j * CHUNK, CHUNK)], sw[b])

    for b in range(NBUF):
        gather(b, b)

    def step(g):
        for b in range(NBUF):
            j = g + b
            gather_done = pltpu.make_async_copy(
                table_hbm.at[idx_v.at[j]], rows[b], sg[b])
            gather_done.wait()
            write(j, b).wait()
            gather(j + NBUF, b)

    pl.loop(0, NSTEPS - NBUF, step=NBUF)(step)

    for b in range(NBUF):
        j = NSTEPS - NBUF + b
        pltpu.make_async_copy(table_hbm.at[idx_v.at[j]], rows[b], sg[b]).wait()
        write(j, b).wait()


@jax.jit
def kernel(x, table):
    x3 = x.reshape(NW, NSTEPS, CHUNK).astype(jnp.int32)
    run = functools.partial(
        pl.kernel,
        out_type=jax.ShapeDtypeStruct((TOTAL, EMBED_DIM), jnp.float32),
        mesh=plsc.VectorSubcoreMesh(core_axis_name="c", subcore_axis_name="s"),
        scratch_types=[
            pltpu.VMEM((NSTEPS, CHUNK), jnp.int32),
            [pltpu.VMEM((CHUNK, EMBED_DIM), jnp.float32)] * NBUF,
            [pltpu.SemaphoreType.DMA] * NBUF,
            [pltpu.SemaphoreType.DMA] * NBUF,
        ],
    )(_body)
    out = run(x3, table)
    return out.reshape(BATCH, SEQ_LEN, EMBED_DIM)


# speedup vs baseline: 9.3001x; 1.0029x over previous
import functools

import jax
import jax.numpy as jnp
from jax import lax
from jax.experimental import pallas as pl
from jax.experimental.pallas import tpu as pltpu
from jax.experimental.pallas import tpu_sc as plsc

VOCAB = 100000
EMBED_DIM = 128
BATCH = 4096
SEQ_LEN = 200

NC = 2
NS = 16
NW = NC * NS

TOTAL = BATCH * SEQ_LEN
PER_W = TOTAL // NW
CHUNK = 128
NSTEPS = PER_W // CHUNK


NBUF = 4


def _body(x_hbm, table_hbm, out_hbm, idx_v, rows, sg, sw):
    wid = lax.axis_index("s") * NC + lax.axis_index("c")
    base = wid * PER_W
    pltpu.sync_copy(x_hbm.at[wid], idx_v)

    def gather(j, b):
        pltpu.async_copy(table_hbm.at[idx_v.at[j]], rows[b], sg[b])

    def write(j, b):
        return pltpu.async_copy(
            rows[b], out_hbm.at[pl.ds(base + j * CHUNK, CHUNK)], sw[b])

    def wait_gather(j, b):
        pltpu.make_async_copy(table_hbm.at[idx_v.at[j]], rows[b], sg[b]).wait()

    def wait_write(j, b):
        pltpu.make_async_copy(
            rows[b], out_hbm.at[pl.ds(base + j * CHUNK, CHUNK)], sw[b]).wait()

    for b in range(NBUF):
        gather(b, b)
    wait_gather(0, 0)
    write(0, 0)

    def step(g):
        for b in range(NBUF):
            j = g + b + 1
            bj = (b + 1) % NBUF
            wait_gather(j, bj)
            write(j, bj)
            wait_write(j - 1, b)
            gather(j - 1 + NBUF, b)

    pl.loop(0, NSTEPS - NBUF, step=NBUF)(step)

    for j in range(NSTEPS - NBUF + 1, NSTEPS):
        wait_gather(j, j % NBUF)
        write(j, j % NBUF)
        wait_write(j - 1, (j - 1) % NBUF)
    wait_write(NSTEPS - 1, (NSTEPS - 1) % NBUF)


@jax.jit
def kernel(x, table):
    x3 = x.reshape(NW, NSTEPS, CHUNK).astype(jnp.int32)
    run = functools.partial(
        pl.kernel,
        out_type=jax.ShapeDtypeStruct((TOTAL, EMBED_DIM), jnp.float32),
        mesh=plsc.VectorSubcoreMesh(core_axis_name="c", subcore_axis_name="s"),
        scratch_types=[
            pltpu.VMEM((NSTEPS, CHUNK), jnp.int32),
            [pltpu.VMEM((CHUNK, EMBED_DIM), jnp.float32)] * NBUF,
            [pltpu.SemaphoreType.DMA] * NBUF,
            [pltpu.SemaphoreType.DMA] * NBUF,
        ],
    )(_body)
    out = run(x3, table)
    return out.reshape(BATCH, SEQ_LEN, EMBED_DIM)
